# baseline (speedup 1.0000x reference)
import jax
import jax.numpy as jnp
from jax.experimental import pallas as pl

NUM_CLASSES = 1000
TOTAL = 16384
BLOCK_CLS = 40


TB = TOTAL // 2


def _onehot_t_block(x_ref, out_ref):
    c0 = pl.program_id(0) * BLOCK_CLS
    cls = jax.lax.broadcasted_iota(jnp.int32, (BLOCK_CLS, TB), 0) + c0
    xv = x_ref[0:1, :]
    out_ref[...] = (xv == cls).astype(jnp.float32)


def kernel(x):
    x2d = x.astype(jnp.int32).reshape(1, TOTAL)
    out_t = pl.pallas_call(
        _onehot_t_block,
        grid=(NUM_CLASSES // BLOCK_CLS, 2),
        in_specs=[pl.BlockSpec((1, TB), lambda i, j: (0, j))],
        out_specs=pl.BlockSpec((BLOCK_CLS, TB), lambda i, j: (i, j)),
        out_shape=jax.ShapeDtypeStruct((NUM_CLASSES, TOTAL), jnp.float32),
    )(x2d)
    return out_t.T


# speedup vs baseline: 1.7048x; 1.7048x over previous
import jax
import jax.numpy as jnp
from jax.experimental import pallas as pl

NUM_CLASSES = 1000
TOTAL = 16384
BLOCK_CLS = 40


def _onehot_t_block(x_ref, out_ref):
    c0 = pl.program_id(0) * BLOCK_CLS
    cls = jax.lax.broadcasted_iota(jnp.int32, (BLOCK_CLS, TOTAL), 0) + c0
    xv = x_ref[0:1, :]
    out_ref[...] = (xv == cls).astype(jnp.float32)


def kernel(x):
    x2d = x.astype(jnp.int32).reshape(1, TOTAL)
    out_t = pl.pallas_call(
        _onehot_t_block,
        grid=(NUM_CLASSES // BLOCK_CLS,),
        in_specs=[pl.BlockSpec((1, TOTAL), lambda i: (0, 0))],
        out_specs=pl.BlockSpec((BLOCK_CLS, TOTAL), lambda i: (i, 0)),
        out_shape=jax.ShapeDtypeStruct((NUM_CLASSES, TOTAL), jnp.float32),
    )(x2d)
    return out_t.T
